# Initial kernel scaffold; baseline (speedup 1.0000x reference)
import jax
import jax.numpy as jnp
from jax.experimental import pallas as pl


def kernel(x, W_g, b_g, W1, b1, W2, b2):
    raise NotImplementedError("write your pallas kernel here")


# speedup vs baseline: 14.4716x; 14.4716x over previous
import functools

import jax
import jax.numpy as jnp
from jax.experimental import pallas as pl
from jax.experimental.pallas import tpu as pltpu


def _router_kernel(x_ref, wg_ref, bg_ref, idx_ref, wts_ref, *, n_experts):
    pooled = jnp.mean(x_ref[...], axis=0, keepdims=True)
    logits = jnp.dot(pooled, wg_ref[...],
                     preferred_element_type=jnp.float32) + bg_ref[...]
    iota = jax.lax.broadcasted_iota(jnp.int32, logits.shape, 1)
    neg = jnp.finfo(jnp.float32).min
    m1 = jnp.max(logits, axis=1, keepdims=True)
    i1 = jnp.min(jnp.where(logits == m1, iota, n_experts),
                 axis=1, keepdims=True)
    masked = jnp.where(iota == i1, neg, logits)
    m2 = jnp.max(masked, axis=1, keepdims=True)
    i2 = jnp.min(jnp.where(masked == m2, iota, n_experts),
                 axis=1, keepdims=True)
    e2 = jnp.exp(m2 - m1)
    z = 1.0 + e2
    idx_ref[0:1, 0:1] = i1
    idx_ref[0:1, 1:2] = i2
    wts_ref[0:1, 0:1] = 1.0 / z
    wts_ref[0:1, 1:2] = e2 / z


def _expert_kernel(idx_ref, w_ref, x_ref, w1_ref, b1_ref, w2_ref, b2_ref,
                   out_ref, *, top_k):
    p = pl.program_id(0)
    f = pl.program_id(1)
    w = w_ref[p]
    h = jnp.dot(x_ref[...], w1_ref[...],
                preferred_element_type=jnp.float32) + b1_ref[...]
    h = 0.5 * h * (1.0 + jax.lax.erf(h * 0.7071067811865476))
    y = jnp.dot(h, w2_ref[...], preferred_element_type=jnp.float32)

    @pl.when((p % top_k == 0) & (f == 0))
    def _init():
        out_ref[...] = jnp.zeros_like(out_ref)

    out_ref[...] += w * y

    @pl.when(f == 0)
    def _bias():
        out_ref[...] += w * b2_ref[...]


def kernel(x, W_g, b_g, W1, b1, W2, b2):
    B, S, D = x.shape
    E = W_g.shape[1]
    D_FF = W1.shape[2]
    TOP_K = 2
    F_BLK = 512

    idx, wts = pl.pallas_call(
        functools.partial(_router_kernel, n_experts=E),
        grid=(B,),
        in_specs=[
            pl.BlockSpec((None, S, D), lambda b: (b, 0, 0)),
            pl.BlockSpec((D, E), lambda b: (0, 0)),
            pl.BlockSpec((1, E), lambda b: (0, 0)),
        ],
        out_specs=[
            pl.BlockSpec((None, 1, TOP_K), lambda b: (b, 0, 0)),
            pl.BlockSpec((None, 1, TOP_K), lambda b: (b, 0, 0)),
        ],
        out_shape=[
            jax.ShapeDtypeStruct((B, 1, TOP_K), jnp.int32),
            jax.ShapeDtypeStruct((B, 1, TOP_K), jnp.float32),
        ],
    )(x, W_g, b_g.reshape(1, E))

    idx_flat = idx.reshape(-1)
    wts_flat = wts.reshape(-1)

    grid_spec = pltpu.PrefetchScalarGridSpec(
        num_scalar_prefetch=2,
        grid=(B * TOP_K, D_FF // F_BLK),
        in_specs=[
            pl.BlockSpec((None, S, D),
                         lambda p, f, idx, w: (p // TOP_K, 0, 0)),
            pl.BlockSpec((None, D, F_BLK),
                         lambda p, f, idx, w: (idx[p], 0, f)),
            pl.BlockSpec((None, 1, F_BLK),
                         lambda p, f, idx, w: (idx[p], 0, f)),
            pl.BlockSpec((None, F_BLK, D),
                         lambda p, f, idx, w: (idx[p], f, 0)),
            pl.BlockSpec((None, 1, D),
                         lambda p, f, idx, w: (idx[p], 0, 0)),
        ],
        out_specs=pl.BlockSpec((None, S, D),
                               lambda p, f, idx, w: (p // TOP_K, 0, 0)),
    )

    out = pl.pallas_call(
        functools.partial(_expert_kernel, top_k=TOP_K),
        grid_spec=grid_spec,
        out_shape=jax.ShapeDtypeStruct((B, S, D), jnp.float32),
    )(idx_flat, wts_flat, x, W1, b1.reshape(E, 1, D_FF), W2,
      b2.reshape(E, 1, D))
    return out


# speedup vs baseline: 16.1669x; 1.1171x over previous
import functools

import jax
import jax.numpy as jnp
from jax.experimental import pallas as pl
from jax.experimental.pallas import tpu as pltpu


def _router_kernel(x_ref, wg_ref, bg_ref, idx_ref, wts_ref, *, n_experts):
    pooled = jnp.mean(x_ref[...], axis=0, keepdims=True)
    logits = jnp.dot(pooled, wg_ref[...],
                     preferred_element_type=jnp.float32) + bg_ref[...]
    iota = jax.lax.broadcasted_iota(jnp.int32, logits.shape, 1)
    neg = jnp.finfo(jnp.float32).min
    m1 = jnp.max(logits, axis=1, keepdims=True)
    i1 = jnp.min(jnp.where(logits == m1, iota, n_experts),
                 axis=1, keepdims=True)
    masked = jnp.where(iota == i1, neg, logits)
    m2 = jnp.max(masked, axis=1, keepdims=True)
    i2 = jnp.min(jnp.where(masked == m2, iota, n_experts),
                 axis=1, keepdims=True)
    e2 = jnp.exp(m2 - m1)
    z = 1.0 + e2
    idx_ref[0:1, 0:1] = i1
    idx_ref[0:1, 1:2] = i2
    wts_ref[0:1, 0:1] = 1.0 / z
    wts_ref[0:1, 1:2] = e2 / z


def _expert_kernel(idx_ref, w_ref, x_ref, w1_ref, b1_ref, w2_ref, b2_ref,
                   out_ref, *, top_k):
    p = pl.program_id(0)
    f = pl.program_id(1)
    w = w_ref[p]
    h = jnp.dot(x_ref[...], w1_ref[...],
                preferred_element_type=jnp.float32,
                precision=jax.lax.Precision.DEFAULT) + b1_ref[...]
    h = 0.5 * h * (1.0 + jax.lax.erf(h * 0.7071067811865476))
    y = jnp.dot(h, w * w2_ref[...], preferred_element_type=jnp.float32,
                precision=jax.lax.Precision.DEFAULT)

    @pl.when((p % top_k == 0) & (f == 0))
    def _init():
        out_ref[...] = jnp.zeros_like(out_ref)

    out_ref[...] += y

    @pl.when(f == 0)
    def _bias():
        out_ref[...] += w * b2_ref[...]


def kernel(x, W_g, b_g, W1, b1, W2, b2):
    B, S, D = x.shape
    E = W_g.shape[1]
    D_FF = W1.shape[2]
    TOP_K = 2
    F_BLK = 1536

    idx, wts = pl.pallas_call(
        functools.partial(_router_kernel, n_experts=E),
        grid=(B,),
        in_specs=[
            pl.BlockSpec((None, S, D), lambda b: (b, 0, 0)),
            pl.BlockSpec((D, E), lambda b: (0, 0)),
            pl.BlockSpec((1, E), lambda b: (0, 0)),
        ],
        out_specs=[
            pl.BlockSpec((None, 1, TOP_K), lambda b: (b, 0, 0)),
            pl.BlockSpec((None, 1, TOP_K), lambda b: (b, 0, 0)),
        ],
        out_shape=[
            jax.ShapeDtypeStruct((B, 1, TOP_K), jnp.int32),
            jax.ShapeDtypeStruct((B, 1, TOP_K), jnp.float32),
        ],
    )(x, W_g, b_g.reshape(1, E))

    idx_flat = idx.reshape(-1)
    wts_flat = wts.reshape(-1)

    grid_spec = pltpu.PrefetchScalarGridSpec(
        num_scalar_prefetch=2,
        grid=(B * TOP_K, D_FF // F_BLK),
        in_specs=[
            pl.BlockSpec((None, S, D),
                         lambda p, f, idx, w: (p // TOP_K, 0, 0)),
            pl.BlockSpec((None, D, F_BLK),
                         lambda p, f, idx, w: (idx[p], 0, f)),
            pl.BlockSpec((None, 1, F_BLK),
                         lambda p, f, idx, w: (idx[p], 0, f)),
            pl.BlockSpec((None, F_BLK, D),
                         lambda p, f, idx, w: (idx[p], f, 0)),
            pl.BlockSpec((None, 1, D),
                         lambda p, f, idx, w: (idx[p], 0, 0)),
        ],
        out_specs=pl.BlockSpec((None, S, D),
                               lambda p, f, idx, w: (p // TOP_K, 0, 0)),
    )

    out = pl.pallas_call(
        functools.partial(_expert_kernel, top_k=TOP_K),
        grid_spec=grid_spec,
        out_shape=jax.ShapeDtypeStruct((B, S, D), jnp.float32),
    )(idx_flat, wts_flat, x, W1, b1.reshape(E, 1, D_FF), W2,
      b2.reshape(E, 1, D))
    return out


# speedup vs baseline: 16.3420x; 1.0108x over previous
import functools

import jax
import jax.numpy as jnp
from jax.experimental import pallas as pl
from jax.experimental.pallas import tpu as pltpu


def _router_kernel(x_ref, wg_ref, bg_ref, idx_ref, wts_ref, *, n_experts):
    pooled = jnp.mean(x_ref[...], axis=0, keepdims=True)
    logits = jnp.dot(pooled, wg_ref[...],
                     preferred_element_type=jnp.float32) + bg_ref[...]
    iota = jax.lax.broadcasted_iota(jnp.int32, logits.shape, 1)
    neg = jnp.finfo(jnp.float32).min
    m1 = jnp.max(logits, axis=1, keepdims=True)
    i1 = jnp.min(jnp.where(logits == m1, iota, n_experts),
                 axis=1, keepdims=True)
    masked = jnp.where(iota == i1, neg, logits)
    m2 = jnp.max(masked, axis=1, keepdims=True)
    i2 = jnp.min(jnp.where(masked == m2, iota, n_experts),
                 axis=1, keepdims=True)
    e2 = jnp.exp(m2 - m1)
    z = 1.0 + e2
    idx_ref[0:1, 0:1] = i1
    idx_ref[0:1, 1:2] = i2
    wts_ref[0:1, 0:1] = 1.0 / z
    wts_ref[0:1, 1:2] = e2 / z


def _expert_kernel(idx_ref, w_ref, x_ref, w1_ref, b1_ref, w2_ref, b2_ref,
                   out_ref, *, top_k):
    b = pl.program_id(0)
    k = pl.program_id(1)
    f = pl.program_id(2)
    w = w_ref[b * top_k + k]
    h = jnp.dot(x_ref[...], w1_ref[...],
                preferred_element_type=jnp.float32) + b1_ref[...]
    h = h * (1.0 + jax.lax.erf(h * 0.7071067811865476))
    y = jnp.dot(h, (0.5 * w) * w2_ref[...],
                preferred_element_type=jnp.float32)

    @pl.when((k == 0) & (f == 0))
    def _init():
        out_ref[...] = jnp.zeros_like(out_ref)

    out_ref[...] += y

    @pl.when(f == 0)
    def _bias():
        out_ref[...] += w * b2_ref[...]


def kernel(x, W_g, b_g, W1, b1, W2, b2):
    B, S, D = x.shape
    E = W_g.shape[1]
    D_FF = W1.shape[2]
    TOP_K = 2
    F_BLK = 1536

    idx, wts = pl.pallas_call(
        functools.partial(_router_kernel, n_experts=E),
        grid=(B,),
        in_specs=[
            pl.BlockSpec((None, S, D), lambda b: (b, 0, 0)),
            pl.BlockSpec((D, E), lambda b: (0, 0)),
            pl.BlockSpec((1, E), lambda b: (0, 0)),
        ],
        out_specs=[
            pl.BlockSpec((None, 1, TOP_K), lambda b: (b, 0, 0)),
            pl.BlockSpec((None, 1, TOP_K), lambda b: (b, 0, 0)),
        ],
        out_shape=[
            jax.ShapeDtypeStruct((B, 1, TOP_K), jnp.int32),
            jax.ShapeDtypeStruct((B, 1, TOP_K), jnp.float32),
        ],
    )(x, W_g, b_g.reshape(1, E))

    idx_flat = idx.reshape(-1)
    wts_flat = wts.reshape(-1)

    grid_spec = pltpu.PrefetchScalarGridSpec(
        num_scalar_prefetch=2,
        grid=(B, TOP_K, D_FF // F_BLK),
        in_specs=[
            pl.BlockSpec((None, S, D),
                         lambda b, k, f, idx, w: (b, 0, 0)),
            pl.BlockSpec((None, D, F_BLK),
                         lambda b, k, f, idx, w: (idx[b * 2 + k], 0, f)),
            pl.BlockSpec((None, 1, F_BLK),
                         lambda b, k, f, idx, w: (idx[b * 2 + k], 0, f)),
            pl.BlockSpec((None, F_BLK, D),
                         lambda b, k, f, idx, w: (idx[b * 2 + k], f, 0)),
            pl.BlockSpec((None, 1, D),
                         lambda b, k, f, idx, w: (idx[b * 2 + k], 0, 0)),
        ],
        out_specs=pl.BlockSpec((None, S, D),
                               lambda b, k, f, idx, w: (b, 0, 0)),
    )

    out = pl.pallas_call(
        functools.partial(_expert_kernel, top_k=TOP_K),
        grid_spec=grid_spec,
        out_shape=jax.ShapeDtypeStruct((B, S, D), jnp.float32),
        compiler_params=pltpu.CompilerParams(
            dimension_semantics=("parallel", "arbitrary", "arbitrary")),
    )(idx_flat, wts_flat, x, W1, b1.reshape(E, 1, D_FF), W2,
      b2.reshape(E, 1, D))
    return out


# speedup vs baseline: 16.4537x; 1.0068x over previous
import functools

import jax
import jax.numpy as jnp
from jax.experimental import pallas as pl
from jax.experimental.pallas import tpu as pltpu


def _router_kernel(x_ref, wg_ref, bg_ref, idx_ref, wts_ref, *, n_experts):
    pooled = jnp.mean(x_ref[...], axis=0, keepdims=True)
    logits = jnp.dot(pooled, wg_ref[...],
                     preferred_element_type=jnp.float32) + bg_ref[...]
    iota = jax.lax.broadcasted_iota(jnp.int32, logits.shape, 1)
    neg = jnp.finfo(jnp.float32).min
    m1 = jnp.max(logits, axis=1, keepdims=True)
    i1 = jnp.min(jnp.where(logits == m1, iota, n_experts),
                 axis=1, keepdims=True)
    masked = jnp.where(iota == i1, neg, logits)
    m2 = jnp.max(masked, axis=1, keepdims=True)
    i2 = jnp.min(jnp.where(masked == m2, iota, n_experts),
                 axis=1, keepdims=True)
    e2 = jnp.exp(m2 - m1)
    z = 1.0 + e2
    idx_ref[0:1, 0:1] = i1
    idx_ref[0:1, 1:2] = i2
    wts_ref[0:1, 0:1] = 1.0 / z
    wts_ref[0:1, 1:2] = e2 / z


def _expert_kernel(idx_ref, w_ref, x_ref, w1_ref, b1_ref, w2_ref, b2_ref,
                   out_ref, *, top_k, f_blk, n_f):
    b = pl.program_id(0)
    k = pl.program_id(1)
    f = pl.program_id(2)
    w = w_ref[b * top_k + k]
    c = 0.7071067811865476
    x = x_ref[...]

    @pl.when((k == 0) & (f == 0))
    def _init():
        out_ref[...] = jnp.broadcast_to(w * b2_ref[...], out_ref.shape)

    @pl.when((k != 0) & (f == 0))
    def _bias():
        out_ref[...] += w * b2_ref[...]

    for ft in range(n_f):
        lo, hi = ft * f_blk, (ft + 1) * f_blk
        hs = jnp.dot(x, c * w1_ref[:, lo:hi],
                     preferred_element_type=jnp.float32) + c * b1_ref[:, lo:hi]
        act = hs + hs * jax.lax.erf(hs)
        out_ref[...] += jnp.dot(act, (c * w) * w2_ref[lo:hi, :],
                                preferred_element_type=jnp.float32)


def kernel(x, W_g, b_g, W1, b1, W2, b2):
    B, S, D = x.shape
    E = W_g.shape[1]
    D_FF = W1.shape[2]
    TOP_K = 2
    F_BLK = 512

    idx, wts = pl.pallas_call(
        functools.partial(_router_kernel, n_experts=E),
        grid=(B,),
        in_specs=[
            pl.BlockSpec((None, S, D), lambda b: (b, 0, 0)),
            pl.BlockSpec((D, E), lambda b: (0, 0)),
            pl.BlockSpec((1, E), lambda b: (0, 0)),
        ],
        out_specs=[
            pl.BlockSpec((None, 1, TOP_K), lambda b: (b, 0, 0)),
            pl.BlockSpec((None, 1, TOP_K), lambda b: (b, 0, 0)),
        ],
        out_shape=[
            jax.ShapeDtypeStruct((B, 1, TOP_K), jnp.int32),
            jax.ShapeDtypeStruct((B, 1, TOP_K), jnp.float32),
        ],
    )(x, W_g, b_g.reshape(1, E))

    idx_flat = idx.reshape(-1)
    wts_flat = wts.reshape(-1)

    F_HALF = D_FF // 2
    grid_spec = pltpu.PrefetchScalarGridSpec(
        num_scalar_prefetch=2,
        grid=(B, TOP_K, 2),
        in_specs=[
            pl.BlockSpec((None, S, D),
                         lambda b, k, f, idx, w: (b, 0, 0)),
            pl.BlockSpec((None, D, F_HALF),
                         lambda b, k, f, idx, w: (idx[b * 2 + k], 0, f)),
            pl.BlockSpec((None, 1, F_HALF),
                         lambda b, k, f, idx, w: (idx[b * 2 + k], 0, f)),
            pl.BlockSpec((None, F_HALF, D),
                         lambda b, k, f, idx, w: (idx[b * 2 + k], f, 0)),
            pl.BlockSpec((None, 1, D),
                         lambda b, k, f, idx, w: (idx[b * 2 + k], 0, 0)),
        ],
        out_specs=pl.BlockSpec((None, S, D),
                               lambda b, k, f, idx, w: (b, 0, 0)),
    )

    out = pl.pallas_call(
        functools.partial(_expert_kernel, top_k=TOP_K, f_blk=F_BLK,
                          n_f=F_HALF // F_BLK),
        grid_spec=grid_spec,
        out_shape=jax.ShapeDtypeStruct((B, S, D), jnp.float32),
        compiler_params=pltpu.CompilerParams(
            dimension_semantics=("parallel", "arbitrary", "arbitrary")),
    )(idx_flat, wts_flat, x, W1, b1.reshape(E, 1, D_FF), W2,
      b2.reshape(E, 1, D))
    return out


# speedup vs baseline: 16.6322x; 1.0108x over previous
import functools

import jax
import jax.numpy as jnp
from jax.experimental import pallas as pl
from jax.experimental.pallas import tpu as pltpu


def _router_kernel(x_ref, wg_ref, bg_ref, idx_ref, wts_ref, *, n_experts):
    pooled = jnp.mean(x_ref[...], axis=0, keepdims=True)
    logits = jnp.dot(pooled, wg_ref[...],
                     preferred_element_type=jnp.float32) + bg_ref[...]
    iota = jax.lax.broadcasted_iota(jnp.int32, logits.shape, 1)
    neg = jnp.finfo(jnp.float32).min
    m1 = jnp.max(logits, axis=1, keepdims=True)
    i1 = jnp.min(jnp.where(logits == m1, iota, n_experts),
                 axis=1, keepdims=True)
    masked = jnp.where(iota == i1, neg, logits)
    m2 = jnp.max(masked, axis=1, keepdims=True)
    i2 = jnp.min(jnp.where(masked == m2, iota, n_experts),
                 axis=1, keepdims=True)
    e2 = jnp.exp(m2 - m1)
    z = 1.0 + e2
    idx_ref[0:1, 0:1] = i1
    idx_ref[0:1, 1:2] = i2
    wts_ref[0:1, 0:1] = 1.0 / z
    wts_ref[0:1, 1:2] = e2 / z


def _expert_kernel(idx_ref, w_ref, x_ref, w1_ref, b1_ref, w2_ref, b2_ref,
                   out_ref, *, top_k, f_blk, n_f):
    b = pl.program_id(0)
    k = pl.program_id(1)
    f = pl.program_id(2)
    w = w_ref[b * top_k + k]
    c = 0.7071067811865476
    x = x_ref[...]

    @pl.when((k == 0) & (f == 0))
    def _init():
        out_ref[...] = jnp.broadcast_to(w * b2_ref[...], out_ref.shape)

    @pl.when((k != 0) & (f == 0))
    def _bias():
        out_ref[...] += w * b2_ref[...]

    for ft in range(n_f):
        lo, hi = ft * f_blk, (ft + 1) * f_blk
        hs = jnp.dot(x, c * w1_ref[:, lo:hi],
                     preferred_element_type=jnp.float32) + c * b1_ref[:, lo:hi]
        act = hs + hs * jax.lax.erf(hs)
        out_ref[...] += jnp.dot(act, (c * w) * w2_ref[lo:hi, :],
                                preferred_element_type=jnp.float32)


def kernel(x, W_g, b_g, W1, b1, W2, b2):
    B, S, D = x.shape
    E = W_g.shape[1]
    D_FF = W1.shape[2]
    TOP_K = 2
    F_BLK = 768

    idx, wts = pl.pallas_call(
        functools.partial(_router_kernel, n_experts=E),
        grid=(B,),
        in_specs=[
            pl.BlockSpec((None, S, D), lambda b: (b, 0, 0)),
            pl.BlockSpec((D, E), lambda b: (0, 0)),
            pl.BlockSpec((1, E), lambda b: (0, 0)),
        ],
        out_specs=[
            pl.BlockSpec((None, 1, TOP_K), lambda b: (b, 0, 0)),
            pl.BlockSpec((None, 1, TOP_K), lambda b: (b, 0, 0)),
        ],
        out_shape=[
            jax.ShapeDtypeStruct((B, 1, TOP_K), jnp.int32),
            jax.ShapeDtypeStruct((B, 1, TOP_K), jnp.float32),
        ],
    )(x, W_g, b_g.reshape(1, E))

    idx_flat = idx.reshape(-1)
    wts_flat = wts.reshape(-1)

    F_HALF = D_FF // 2
    grid_spec = pltpu.PrefetchScalarGridSpec(
        num_scalar_prefetch=2,
        grid=(B, TOP_K, 2),
        in_specs=[
            pl.BlockSpec((None, S, D),
                         lambda b, k, f, idx, w: (b, 0, 0)),
            pl.BlockSpec((None, D, F_HALF),
                         lambda b, k, f, idx, w: (idx[b * 2 + k], 0, f)),
            pl.BlockSpec((None, 1, F_HALF),
                         lambda b, k, f, idx, w: (idx[b * 2 + k], 0, f)),
            pl.BlockSpec((None, F_HALF, D),
                         lambda b, k, f, idx, w: (idx[b * 2 + k], f, 0)),
            pl.BlockSpec((None, 1, D),
                         lambda b, k, f, idx, w: (idx[b * 2 + k], 0, 0)),
        ],
        out_specs=pl.BlockSpec((None, S, D),
                               lambda b, k, f, idx, w: (b, 0, 0)),
    )

    out = pl.pallas_call(
        functools.partial(_expert_kernel, top_k=TOP_K, f_blk=F_BLK,
                          n_f=F_HALF // F_BLK),
        grid_spec=grid_spec,
        out_shape=jax.ShapeDtypeStruct((B, S, D), jnp.float32),
        compiler_params=pltpu.CompilerParams(
            dimension_semantics=("parallel", "arbitrary", "arbitrary")),
    )(idx_flat, wts_flat, x, W1, b1.reshape(E, 1, D_FF), W2,
      b2.reshape(E, 1, D))
    return out
